# Initial kernel scaffold; baseline (speedup 1.0000x reference)
import jax
import jax.numpy as jnp
from jax.experimental import pallas as pl


def kernel(features, adj_indices, adj_values, index):
    raise NotImplementedError("write your pallas kernel here")


# speedup vs baseline: 3.4446x; 3.4446x over previous
import functools

import jax
import jax.numpy as jnp
from jax import lax
from jax.experimental import pallas as pl
from jax.experimental.pallas import tpu as pltpu
from jax.experimental.pallas import tpu_sc as plsc

N_NODES = 10000
N_EDGES = 320000
D_FEAT = 128

NC = 2
NS = 16
NW = NC * NS
EPW = N_EDGES // NW
CHUNK = 80
NCHUNK = EPW // CHUNK
ROWS_PER_TILE = N_NODES // NS
ZROWS = 125
LANES = 16
NSEG = D_FEAT // LANES


def _sc_spmm(features, row_i32, col_i32, values):
    mesh = plsc.VectorSubcoreMesh(core_axis_name="c", subcore_axis_name="s")

    @functools.partial(
        pl.kernel,
        out_type=jax.ShapeDtypeStruct((NC, N_NODES, D_FEAT), jnp.float32),
        mesh=mesh,
        scratch_types=[
            pltpu.VMEM((CHUNK,), jnp.int32),
            pltpu.VMEM((CHUNK,), jnp.int32),
            pltpu.VMEM((CHUNK,), jnp.float32),
            pltpu.VMEM((CHUNK, D_FEAT), jnp.float32),
            pltpu.VMEM((ZROWS, D_FEAT), jnp.float32),
            pltpu.VMEM_SHARED((N_NODES, D_FEAT), jnp.float32),
            pltpu.SemaphoreType.DMA,
        ],
        compiler_params=pltpu.CompilerParams(use_tc_tiling_on_sc=False),
    )
    def body(feat_hbm, row_hbm, col_hbm, val_hbm, out_hbm,
             colv, rowv, valv, rows, zbuf, accum, sem):
        cid = lax.axis_index("c")
        sid = lax.axis_index("s")
        wid = cid * NS + sid

        zero16 = jnp.zeros((LANES,), jnp.float32)

        def zfill(i, _):
            for k in range(NSEG):
                zbuf[i, pl.ds(k * LANES, LANES)] = zero16
            return 0

        lax.fori_loop(0, ZROWS, zfill, 0)
        t0 = sid * ROWS_PER_TILE
        for j in range(ROWS_PER_TILE // ZROWS):
            pltpu.sync_copy(zbuf, accum.at[pl.ds(t0 + j * ZROWS, ZROWS), :])
        plsc.subcore_barrier()

        ebase = wid * EPW

        def chunk_body(c, _):
            base = ebase + c * CHUNK
            pltpu.sync_copy(col_hbm.at[pl.ds(base, CHUNK)], colv)
            pltpu.sync_copy(row_hbm.at[pl.ds(base, CHUNK)], rowv)
            pltpu.sync_copy(val_hbm.at[pl.ds(base, CHUNK)], valv)
            pltpu.async_copy(feat_hbm.at[colv], rows, sem).wait()

            def scale(g, _):
                vv = valv[pl.ds(g * LANES, LANES)]
                for j in range(LANES):
                    v = vv[j]
                    e = g * LANES + j
                    for k in range(NSEG):
                        rows[e, pl.ds(k * LANES, LANES)] = (
                            rows[e, pl.ds(k * LANES, LANES)] * v
                        )
                return 0

            lax.fori_loop(0, CHUNK // LANES, scale, 0)
            pltpu.sync_copy(rows, accum.at[rowv], add=True)
            return 0

        lax.fori_loop(0, NCHUNK, chunk_body, 0)
        plsc.subcore_barrier()

        pltpu.sync_copy(
            accum.at[pl.ds(t0, ROWS_PER_TILE), :],
            out_hbm.at[cid, pl.ds(t0, ROWS_PER_TILE), :],
        )

    return body(features, row_i32, col_i32, values)


def _tc_combine(partials):
    def add_body(p_ref, o_ref):
        o_ref[...] = p_ref[0] + p_ref[1]

    nblk = 10
    blk = N_NODES // nblk
    return pl.pallas_call(
        add_body,
        grid=(nblk,),
        in_specs=[pl.BlockSpec((NC, blk, D_FEAT), lambda i: (0, i, 0))],
        out_specs=pl.BlockSpec((blk, D_FEAT), lambda i: (i, 0)),
        out_shape=jax.ShapeDtypeStruct((N_NODES, D_FEAT), jnp.float32),
    )(partials)


def kernel(features, adj_indices, adj_values, index):
    assert features.shape == (N_NODES, D_FEAT)
    assert adj_indices.shape == (2, N_EDGES)
    row = adj_indices[0].astype(jnp.int32)
    col = adj_indices[1].astype(jnp.int32)
    partials = _sc_spmm(features, row, col, adj_values.astype(jnp.float32))
    return _tc_combine(partials)
